# Initial kernel scaffold; baseline (speedup 1.0000x reference)
import jax
import jax.numpy as jnp
from jax.experimental import pallas as pl


def kernel(input_ids, token_type_ids, word_embeddings, token_type_embeddings):
    raise NotImplementedError("write your pallas kernel here")


# speedup vs baseline: 1.9350x; 1.9350x over previous
import functools

import jax
import jax.numpy as jnp
from jax import lax
from jax.experimental import pallas as pl
from jax.experimental.pallas import tpu as pltpu
from jax.experimental.pallas import tpu_sc as plsc

H = 1024
NC, NS, L = 2, 16, 16
NW = NC * NS
CH = 32


def _sc_embed(ids, tts, word, ttab, *, n_tok):
    tpw = n_tok // NW
    nchunk = tpw // CH
    hpl = H // L
    mesh = plsc.VectorSubcoreMesh(core_axis_name="c", subcore_axis_name="s")

    @functools.partial(
        pl.kernel,
        out_type=jax.ShapeDtypeStruct((n_tok, H), jnp.float32),
        mesh=mesh,
        scratch_types=[
            pltpu.VMEM((tpw,), jnp.int32),
            pltpu.VMEM((tpw + L,), jnp.int32),
            pltpu.VMEM((2 * H,), jnp.float32),
            pltpu.VMEM((CH, H), jnp.float32),
            pltpu.VMEM((CH, H), jnp.float32),
            pltpu.SemaphoreType.DMA,
            pltpu.SemaphoreType.DMA,
            pltpu.SemaphoreType.DMA,
            pltpu.SemaphoreType.DMA,
        ],
    )
    def k(ids_hbm, tts_hbm, word_hbm, ttab_hbm, out_hbm,
          idx_v, toff_v, ttb_v, buf0, buf1, gs0, gs1, os0, os1):
        wid = lax.axis_index("s") * NC + lax.axis_index("c")
        base = wid * tpw
        pltpu.sync_copy(ids_hbm.at[pl.ds(base, tpw)], idx_v)
        pltpu.sync_copy(tts_hbm.at[pl.ds(base, tpw)], toff_v.at[pl.ds(0, tpw)])
        pltpu.sync_copy(ttab_hbm, ttb_v)

        @plsc.parallel_loop(0, tpw, step=L)
        def _mkoff(j):
            toff_v[pl.ds(j, L)] = toff_v[pl.ds(j, L)] * H

        def fire_gather(c, buf, sem):
            pltpu.async_copy(word_hbm.at[idx_v.at[pl.ds(c * CH, CH)]], buf, sem)

        def wait_gather(c, buf, sem):
            pltpu.make_async_copy(
                word_hbm.at[idx_v.at[pl.ds(c * CH, CH)]], buf, sem).wait()

        def fire_out(c, buf, sem):
            pltpu.async_copy(buf, out_hbm.at[pl.ds(base + c * CH, CH)], sem)

        def wait_out(buf, sem):
            pltpu.make_async_copy(buf, out_hbm.at[pl.ds(base, CH)], sem).wait()

        def add_type_rows(c, buf):
            cbase = c * CH

            @plsc.parallel_loop(0, CH, step=1)
            def _add(t):
                toff = toff_v[pl.ds(cbase + t, L)][0]
                for hi in range(hpl):
                    hh = hi * L
                    buf[t, pl.ds(hh, L)] = (
                        buf[t, pl.ds(hh, L)] + ttb_v[pl.ds(toff + hh, L)])

        fire_gather(0, buf0, gs0)

        def body(gp, carry):
            c0 = 2 * gp
            wait_gather(c0, buf0, gs0)

            @pl.when(gp > 0)
            def _():
                wait_out(buf1, os1)

            fire_gather(c0 + 1, buf1, gs1)
            add_type_rows(c0, buf0)
            fire_out(c0, buf0, os0)

            wait_gather(c0 + 1, buf1, gs1)

            @pl.when(gp < nchunk // 2 - 1)
            def _():
                wait_out(buf0, os0)
                fire_gather(c0 + 2, buf0, gs0)

            add_type_rows(c0 + 1, buf1)
            fire_out(c0 + 1, buf1, os1)
            return carry

        lax.fori_loop(0, nchunk // 2, body, 0)
        wait_out(buf0, os0)
        wait_out(buf1, os1)

    return k(ids, tts, word, ttab)


def kernel(input_ids, token_type_ids, word_embeddings, token_type_embeddings):
    b, s = input_ids.shape
    n = b * s
    ids = input_ids.reshape(n).astype(jnp.int32)
    tts = token_type_ids.reshape(n).astype(jnp.int32)
    ttab = token_type_embeddings.reshape(-1)
    out = _sc_embed(ids, tts, word_embeddings, ttab, n_tok=n)
    return out.reshape(b, s, word_embeddings.shape[1])
